# Initial kernel scaffold; baseline (speedup 1.0000x reference)
import jax
import jax.numpy as jnp
from jax.experimental import pallas as pl


def kernel(x, edge_index, edge_attribute, Wn1, bn1, root1, bias1, Wn2, bn2, root2, bias2, Wn3, bn3, root3, bias3):
    raise NotImplementedError("write your pallas kernel here")


# speedup vs baseline: 3.9358x; 3.9358x over previous
import functools

import jax
import jax.numpy as jnp
from jax import lax
from jax.experimental import pallas as pl
from jax.experimental.pallas import tpu as pltpu
from jax.experimental.pallas import tpu_sc as plsc

N = 10000
E = 320000
D = 128

NC = 2
NS = 16
CH = 80
EDGES_PER_CORE = E // NC
EDGES_PER_TILE = EDGES_PER_CORE // NS
NCHUNK = EDGES_PER_TILE // CH
NPAD = 10240
ZROWS = 128
ROWS_PER_TILE = NPAD // NS


def _edge_body(src_hbm, dst_hbm, ea_hbm, clo_hbm, chi_hbm, out0_hbm, out1_hbm,
               src_v, dst_v, ea_v, rows_lo, rows_hi, msg_v, zbuf, acc_sh, sem):
    c = lax.axis_index("c")
    s = lax.axis_index("s")

    def zloop(i, _):
        for k in range(8):
            zbuf[i, pl.ds(k * 16, 16)] = jnp.zeros((16,), jnp.float32)
        return 0
    lax.fori_loop(0, ZROWS, zloop, 0)
    for r in range(ROWS_PER_TILE // ZROWS):
        pltpu.sync_copy(zbuf, acc_sh.at[pl.ds(s * ROWS_PER_TILE + r * ZROWS, ZROWS)])
    plsc.subcore_barrier()

    tile_base = c * EDGES_PER_CORE + s * EDGES_PER_TILE

    def chunk(j, _):
        eb = tile_base + j * CH
        pltpu.sync_copy(src_hbm.at[pl.ds(eb, CH)], src_v)
        pltpu.sync_copy(dst_hbm.at[pl.ds(eb, CH)], dst_v)
        pltpu.sync_copy(ea_hbm.at[pl.ds(eb, CH)], ea_v)
        pltpu.async_copy(clo_hbm.at[src_v], rows_lo, sem).wait()
        pltpu.async_copy(chi_hbm.at[src_v], rows_hi, sem).wait()

        for g in range(CH // 16):
            ea_vec = ea_v[pl.ds(g * 16, 16)]
            for e16 in range(16):
                w = lax.gather(
                    ea_vec, jnp.full((16, 1), e16, jnp.int32),
                    lax.GatherDimensionNumbers(offset_dims=(),
                                               collapsed_slice_dims=(0,),
                                               start_index_map=(0,)),
                    slice_sizes=(1,),
                    mode=lax.GatherScatterMode.PROMISE_IN_BOUNDS)
                e = g * 16 + e16
                for k in range(8):
                    msg_v[e, pl.ds(k * 16, 16)] = w * rows_lo[e, pl.ds(k * 16, 16)]

        pltpu.sync_copy(msg_v, acc_sh.at[dst_v], add=True)
        pltpu.sync_copy(rows_hi, acc_sh.at[dst_v], add=True)
        return 0
    lax.fori_loop(0, NCHUNK, chunk, 0)
    plsc.subcore_barrier()

    @pl.when(c == 0)
    def _():
        for r in range(ROWS_PER_TILE // ZROWS):
            row = s * ROWS_PER_TILE + r * ZROWS
            pltpu.sync_copy(acc_sh.at[pl.ds(row, ZROWS)],
                            out0_hbm.at[pl.ds(row, ZROWS)])

    @pl.when(c == 1)
    def _():
        for r in range(ROWS_PER_TILE // ZROWS):
            row = s * ROWS_PER_TILE + r * ZROWS
            pltpu.sync_copy(acc_sh.at[pl.ds(row, ZROWS)],
                            out1_hbm.at[pl.ds(row, ZROWS)])


_edge_pass = functools.partial(
    pl.kernel,
    out_type=[jax.ShapeDtypeStruct((NPAD, D), jnp.float32)] * 2,
    mesh=plsc.VectorSubcoreMesh(core_axis_name="c", subcore_axis_name="s"),
    scratch_types=[
        pltpu.VMEM((CH,), jnp.int32),
        pltpu.VMEM((CH,), jnp.int32),
        pltpu.VMEM((CH,), jnp.float32),
        pltpu.VMEM((CH, D), jnp.float32),
        pltpu.VMEM((CH, D), jnp.float32),
        pltpu.VMEM((CH, D), jnp.float32),
        pltpu.VMEM((ZROWS, D), jnp.float32),
        pltpu.VMEM_SHARED((NPAD, D), jnp.float32),
        pltpu.SemaphoreType.DMA,
    ],
)(_edge_body)


_RB = 1000
_NB = N // _RB


def _pre_body(x_ref, wn_ref, bn_ref, clo_ref, chi_ref):
    xb = x_ref[pl.ds(pl.program_id(0) * _RB, _RB), :]
    clo_ref[...] = xb * wn_ref[...]
    chi_ref[...] = xb * bn_ref[...]


def _mid_body(a0_ref, a1_ref, h_ref, root_ref, bias_ref, wn_ref, bn_ref,
              h_out, clo_out, chi_out, *, first):
    g = a0_ref[...] + a1_ref[...] + bias_ref[...]
    if first:
        g = g + h_ref[pl.ds(pl.program_id(0) * _RB, _RB), :] * root_ref[...]
    else:
        g = g + jnp.dot(h_ref[...], root_ref[...], preferred_element_type=jnp.float32)
    g = jnp.maximum(g, 0.0)
    h_out[...] = g
    clo_out[...] = jnp.dot(g, wn_ref[...], preferred_element_type=jnp.float32)
    chi_out[...] = jnp.dot(g, bn_ref[...], preferred_element_type=jnp.float32)


def _final_body(a0_ref, a1_ref, h_ref, root_ref, bias_ref, out_ref):
    out_ref[...] = (a0_ref[...] + a1_ref[...] + bias_ref[...]
                    + jnp.dot(h_ref[...], root_ref[...], preferred_element_type=jnp.float32))


def _row_spec(width):
    return pl.BlockSpec((_RB, width), lambda i: (i, 0))


def _full_spec(r, width):
    return pl.BlockSpec((r, width), lambda i: (0, 0))


_ACC = pl.BlockSpec((_RB, D), lambda i: (i, 0))


def _pre(x, wn, bn):
    return pl.pallas_call(
        _pre_body,
        grid=(_NB,),
        in_specs=[_full_spec(N, 1), _full_spec(1, D), _full_spec(1, D)],
        out_specs=[_row_spec(D), _row_spec(D)],
        out_shape=[jax.ShapeDtypeStruct((N, D), jnp.float32)] * 2,
    )(x, wn, bn)


def _mid(acc0, acc1, h, root, bias, wn, bn, *, first):
    hw = h.shape[1]
    return pl.pallas_call(
        functools.partial(_mid_body, first=first),
        grid=(_NB,),
        in_specs=[_ACC, _ACC,
                  _full_spec(N, 1) if first else _row_spec(hw),
                  _full_spec(root.shape[0], D),
                  _full_spec(1, D), _full_spec(D, D), _full_spec(D, D)],
        out_specs=[_row_spec(D)] * 3,
        out_shape=[jax.ShapeDtypeStruct((N, D), jnp.float32)] * 3,
    )(acc0, acc1, h, root, bias, wn, bn)


def _final(acc0, acc1, h, root, bias):
    return pl.pallas_call(
        _final_body,
        grid=(_NB,),
        in_specs=[_ACC, _ACC, _row_spec(D), _full_spec(D, D), _full_spec(1, D)],
        out_specs=_row_spec(D),
        out_shape=jax.ShapeDtypeStruct((N, D), jnp.float32),
    )(acc0, acc1, h, root, bias)


def kernel(x, edge_index, edge_attribute, Wn1, bn1, root1, bias1,
           Wn2, bn2, root2, bias2, Wn3, bn3, root3, bias3):
    src = edge_index[0]
    dst = edge_index[1]
    ea = edge_attribute[:, 0]

    c1lo, c1hi = _pre(x, Wn1, bn1.reshape(1, D))
    a0, a1 = _edge_pass(src, dst, ea, c1lo, c1hi)
    h1, c2lo, c2hi = _mid(a0, a1, x, root1, bias1.reshape(1, D),
                          Wn2.reshape(D, D), bn2.reshape(D, D), first=True)
    a0, a1 = _edge_pass(src, dst, ea, c2lo, c2hi)
    h2, c3lo, c3hi = _mid(a0, a1, h1, root2, bias2.reshape(1, D),
                          Wn3.reshape(D, D), bn3.reshape(D, D), first=False)
    a0, a1 = _edge_pass(src, dst, ea, c3lo, c3hi)
    return _final(a0, a1, h2, root3, bias3.reshape(1, D))


# speedup vs baseline: 6.3753x; 1.6198x over previous
import functools

import jax
import jax.numpy as jnp
from jax import lax
from jax.experimental import pallas as pl
from jax.experimental.pallas import tpu as pltpu
from jax.experimental.pallas import tpu_sc as plsc

N = 10000
E = 320000
D = 128

NC = 2
NS = 16
CH = 80
EDGES_PER_CORE = E // NC
EDGES_PER_TILE = EDGES_PER_CORE // NS
NCHUNK = EDGES_PER_TILE // CH
NCKP = NCHUNK + 1
NBODY = NCKP // 3
NPAD = 10240
ROWS_PER_TILE = NPAD // NS
IB_SRC = 0
IB_DST = NCKP
_DNUMS = lax.GatherDimensionNumbers(offset_dims=(), collapsed_slice_dims=(0,),
                                    start_index_map=(0,))


def _edge_body(pki_hbm, pke_hbm, z_hbm, clo_hbm, chi_hbm, out0_hbm, out1_hbm,
               idx_r, ea_r, b0, b1, b2, b3, acc_sh,
               si0, si1, si2, sg0, sg1, sg2, sg3, ss0, ss1, ss2, ss3):
    c = lax.axis_index("c")
    s = lax.axis_index("s")
    BUF = (b0, b1, b2, b3)
    SI = (si0, si1, si2)
    SG = (sg0, sg1, sg2, sg3)
    SS = (ss0, ss1, ss2, ss3)

    pltpu.sync_copy(z_hbm, acc_sh.at[pl.ds(s * ROWS_PER_TILE, ROWS_PER_TILE)])

    def fire_idx(j, sl):
        pltpu.async_copy(pki_hbm.at[c, s, j], idx_r.at[sl], SI[sl])
        pltpu.async_copy(pke_hbm.at[c, s, j], ea_r.at[sl], SI[sl])

    def wait_idx(j, sl):
        pltpu.make_async_copy(pki_hbm.at[c, s, j], idx_r.at[sl], SI[sl]).wait()
        pltpu.make_async_copy(pke_hbm.at[c, s, j], ea_r.at[sl], SI[sl]).wait()

    def fire_gather(j, sl, r, kind):
        tbl = clo_hbm if kind else chi_hbm
        pltpu.async_copy(tbl.at[idx_r.at[sl, 0]], BUF[r], SG[r])

    def wait_gather(j, sl, r, kind):
        tbl = clo_hbm if kind else chi_hbm
        pltpu.make_async_copy(tbl.at[idx_r.at[sl, 0]], BUF[r], SG[r]).wait()

    def fire_scatter(j, sl, r):
        pltpu.async_copy(BUF[r], acc_sh.at[idx_r.at[sl, 1]], SS[r], add=True)

    def wait_scatter(j, sl, r):
        pltpu.make_async_copy(BUF[r], acc_sh.at[idx_r.at[sl, 1]], SS[r]).wait()

    def compute(sl, r):
        buf = BUF[r]

        def grp(g, _):
            ea_vec = ea_r[sl, g, :]
            for e16 in range(16):
                w = lax.gather(ea_vec, jnp.full((16, 1), e16, jnp.int32),
                               _DNUMS, slice_sizes=(1,),
                               mode=lax.GatherScatterMode.PROMISE_IN_BOUNDS)
                e = g * 16 + e16
                for k in range(8):
                    buf[e, pl.ds(k * 16, 16)] = w * buf[e, pl.ds(k * 16, 16)]
            return 0
        lax.fori_loop(0, CH // 16, grp, 0)

    for j0 in range(3):
        fire_idx(j0, j0)
    wait_idx(0, 0)
    fire_gather(0, 0, 0, 0)
    plsc.subcore_barrier()

    NJOB = 2 * NCKP

    def body(t, _):
        for v in range(12):
            q = 12 * t + v
            jc = q // 2
            vj = v // 2
            kind = v % 2
            r = v % 4
            sl = vj % 3
            wait_gather(jc, sl, r, kind)
            if kind == 1:
                compute(sl, r)
            fire_scatter(jc, sl, r)

            slm = ((v - 3) % 12 // 2) % 3
            rm = (r + 1) % 4

            @pl.when(q >= 3)
            def _(slm=slm, rm=rm):
                wait_scatter(0, slm, rm)
            if kind == 0:
                jn = jc + 1
                sln = ((vj + 1) % 3)

                @pl.when((jn >= 3) & (jn < NCKP))
                def _(jn=jn, sln=sln):
                    fire_idx(jn, sln)
            qn = q + 1
            jn2 = qn // 2
            vn = (v + 1) % 12
            sln2 = (vn // 2) % 3
            rn = (r + 1) % 4
            kn = 1 - kind
            if kind == 1:
                @pl.when(jn2 < NCKP)
                def _(jn2=jn2, sln2=sln2):
                    wait_idx(jn2, sln2)

            @pl.when(qn < NJOB)
            def _(jn2=jn2, sln2=sln2, rn=rn, kn=kn):
                fire_gather(jn2, sln2, rn, kn)
        return 0
    lax.fori_loop(0, NJOB // 12, body, 0)
    for qq in range(NJOB - 3, NJOB):
        vv = qq % 12
        wait_scatter(qq // 2, (vv // 2) % 3, vv % 4)
    plsc.subcore_barrier()

    rows = pl.ds(s * ROWS_PER_TILE, ROWS_PER_TILE)

    @pl.when(c == 0)
    def _():
        pltpu.sync_copy(acc_sh.at[rows], out0_hbm.at[rows])

    @pl.when(c == 1)
    def _():
        pltpu.sync_copy(acc_sh.at[rows], out1_hbm.at[rows])


_edge_pass = functools.partial(
    pl.kernel,
    out_type=[jax.ShapeDtypeStruct((NPAD, D), jnp.float32)] * 2,
    mesh=plsc.VectorSubcoreMesh(core_axis_name="c", subcore_axis_name="s"),
    scratch_types=[
        pltpu.VMEM((3, 2, CH), jnp.int32),
        pltpu.VMEM((3, CH // 16, 16), jnp.float32),
        pltpu.VMEM((CH, D), jnp.float32),
        pltpu.VMEM((CH, D), jnp.float32),
        pltpu.VMEM((CH, D), jnp.float32),
        pltpu.VMEM((CH, D), jnp.float32),
        pltpu.VMEM_SHARED((NPAD, D), jnp.float32),
        pltpu.SemaphoreType.DMA,
        pltpu.SemaphoreType.DMA,
        pltpu.SemaphoreType.DMA,
        pltpu.SemaphoreType.DMA,
        pltpu.SemaphoreType.DMA,
        pltpu.SemaphoreType.DMA,
        pltpu.SemaphoreType.DMA,
        pltpu.SemaphoreType.DMA,
        pltpu.SemaphoreType.DMA,
        pltpu.SemaphoreType.DMA,
        pltpu.SemaphoreType.DMA,
    ],
)(_edge_body)


_RB = 1000
_NB = N // _RB


def _pre_body(x_ref, wn_ref, bn_ref, clo_ref, chi_ref):
    xb = x_ref[pl.ds(pl.program_id(0) * _RB, _RB), :]
    clo_ref[...] = xb * wn_ref[...]
    chi_ref[...] = xb * bn_ref[...]


def _mid_body(a0_ref, a1_ref, h_ref, root_ref, bias_ref, wn_ref, bn_ref,
              h_out, clo_out, chi_out, *, first):
    g = a0_ref[...] + a1_ref[...] + bias_ref[...]
    if first:
        g = g + h_ref[pl.ds(pl.program_id(0) * _RB, _RB), :] * root_ref[...]
    else:
        g = g + jnp.dot(h_ref[...], root_ref[...], preferred_element_type=jnp.float32)
    g = jnp.maximum(g, 0.0)
    h_out[...] = g
    clo_out[...] = jnp.dot(g, wn_ref[...], preferred_element_type=jnp.float32)
    chi_out[...] = jnp.dot(g, bn_ref[...], preferred_element_type=jnp.float32)


def _final_body(a0_ref, a1_ref, h_ref, root_ref, bias_ref, out_ref):
    out_ref[...] = (a0_ref[...] + a1_ref[...] + bias_ref[...]
                    + jnp.dot(h_ref[...], root_ref[...], preferred_element_type=jnp.float32))


def _row_spec(width):
    return pl.BlockSpec((_RB, width), lambda i: (i, 0))


def _full_spec(r, width):
    return pl.BlockSpec((r, width), lambda i: (0, 0))


_ACC = pl.BlockSpec((_RB, D), lambda i: (i, 0))


def _pre(x, wn, bn):
    return pl.pallas_call(
        _pre_body,
        grid=(_NB,),
        in_specs=[_full_spec(N, 1), _full_spec(1, D), _full_spec(1, D)],
        out_specs=[_row_spec(D), _row_spec(D)],
        out_shape=[jax.ShapeDtypeStruct((N, D), jnp.float32)] * 2,
    )(x, wn, bn)


def _mid(acc0, acc1, h, root, bias, wn, bn, *, first):
    hw = h.shape[1]
    return pl.pallas_call(
        functools.partial(_mid_body, first=first),
        grid=(_NB,),
        in_specs=[_ACC, _ACC,
                  _full_spec(N, 1) if first else _row_spec(hw),
                  _full_spec(root.shape[0], D),
                  _full_spec(1, D), _full_spec(D, D), _full_spec(D, D)],
        out_specs=[_row_spec(D)] * 3,
        out_shape=[jax.ShapeDtypeStruct((N, D), jnp.float32)] * 3,
    )(acc0, acc1, h, root, bias, wn, bn)


def _final(acc0, acc1, h, root, bias):
    return pl.pallas_call(
        _final_body,
        grid=(_NB,),
        in_specs=[_ACC, _ACC, _row_spec(D), _full_spec(D, D), _full_spec(1, D)],
        out_specs=_row_spec(D),
        out_shape=jax.ShapeDtypeStruct((N, D), jnp.float32),
    )(acc0, acc1, h, root, bias)


def kernel(x, edge_index, edge_attribute, Wn1, bn1, root1, bias1,
           Wn2, bn2, root2, bias2, Wn3, bn3, root3, bias3):
    src = edge_index[0]
    dst = edge_index[1]
    ea = edge_attribute[:, 0]

    lanes = jnp.arange(CH, dtype=jnp.int32)
    pad_src = jnp.broadcast_to((lanes * 131) % N, (NC, NS, 1, CH))
    pad_dst = jnp.broadcast_to(10232 + (lanes % 8), (NC, NS, 1, CH))
    S = jnp.concatenate([src.reshape(NC, NS, NCHUNK, CH), pad_src], axis=2)
    T = jnp.concatenate([dst.reshape(NC, NS, NCHUNK, CH), pad_dst], axis=2)
    pki = jnp.stack([S, T], axis=3)
    pke = jnp.concatenate(
        [ea.reshape(NC, NS, NCHUNK, CH),
         jnp.zeros((NC, NS, 1, CH), jnp.float32)],
        axis=2).reshape(NC, NS, NCKP, CH // 16, 16)
    z = jnp.zeros((ROWS_PER_TILE, D), jnp.float32)

    c1lo, c1hi = _pre(x, Wn1, bn1.reshape(1, D))
    a0, a1 = _edge_pass(pki, pke, z, c1lo, c1hi)
    h1, c2lo, c2hi = _mid(a0, a1, x, root1, bias1.reshape(1, D),
                          Wn2.reshape(D, D), bn2.reshape(D, D), first=True)
    a0, a1 = _edge_pass(pki, pke, z, c2lo, c2hi)
    h2, c3lo, c3hi = _mid(a0, a1, h1, root2, bias2.reshape(1, D),
                          Wn3.reshape(D, D), bn3.reshape(D, D), first=False)
    a0, a1 = _edge_pass(pki, pke, z, c3lo, c3hi)
    return _final(a0, a1, h2, root3, bias3.reshape(1, D))


# speedup vs baseline: 9.7691x; 1.5323x over previous
import functools

import jax
import jax.numpy as jnp
from jax import lax
from jax.experimental import pallas as pl
from jax.experimental.pallas import tpu as pltpu
from jax.experimental.pallas import tpu_sc as plsc

N = 10000
E = 320000
D = 128

NC = 2
NS = 16
CH = 80
EDGES_PER_CORE = E // NC
EDGES_PER_TILE = EDGES_PER_CORE // NS
NCHUNK = EDGES_PER_TILE // CH
NCKP = NCHUNK + 1
NBODY = NCKP // 3
NPAD = 10240
ROWS_PER_TILE = NPAD // NS
IB_SRC = 0
IB_DST = NCKP
_DNUMS = lax.GatherDimensionNumbers(offset_dims=(), collapsed_slice_dims=(0,),
                                    start_index_map=(0,))


def _edge_body(pki_hbm, pke_hbm, z_hbm, clo_hbm, chi_hbm, out0_hbm, out1_hbm,
               idx_r, ea_r, b0, b1, b2, b3, acc_sh,
               si0, si1, si2, sg0, sg1, sg2, sg3, ss0, ss1, ss2, ss3):
    c = lax.axis_index("c")
    s = lax.axis_index("s")
    BUF = (b0, b1, b2, b3)
    SI = (si0, si1, si2)
    SG = (sg0, sg1, sg2, sg3)
    SS = (ss0, ss1, ss2, ss3)

    pltpu.sync_copy(z_hbm, acc_sh.at[pl.ds(s * ROWS_PER_TILE, ROWS_PER_TILE)])

    def fire_idx(j, sl):
        pltpu.async_copy(pki_hbm.at[c, s, j], idx_r.at[sl], SI[sl])
        pltpu.async_copy(pke_hbm.at[c, s, j], ea_r.at[sl], SI[sl])

    def wait_idx(j, sl):
        pltpu.make_async_copy(pki_hbm.at[c, s, j], idx_r.at[sl], SI[sl]).wait()
        pltpu.make_async_copy(pke_hbm.at[c, s, j], ea_r.at[sl], SI[sl]).wait()

    def fire_gather(j, sl, r, kind):
        tbl = clo_hbm if kind else chi_hbm
        pltpu.async_copy(tbl.at[idx_r.at[sl, 0]], BUF[r], SG[r])

    def wait_gather(j, sl, r, kind):
        tbl = clo_hbm if kind else chi_hbm
        pltpu.make_async_copy(tbl.at[idx_r.at[sl, 0]], BUF[r], SG[r]).wait()

    def fire_scatter(j, sl, r):
        pltpu.async_copy(BUF[r], acc_sh.at[idx_r.at[sl, 1]], SS[r], add=True)

    def wait_scatter(j, sl, r):
        pltpu.make_async_copy(BUF[r], acc_sh.at[idx_r.at[sl, 1]], SS[r]).wait()

    def compute(sl, r):
        buf = BUF[r]

        def grp(g, _):
            ea_vec = ea_r[sl, g, :]
            for e16 in range(16):
                w = lax.gather(ea_vec, jnp.full((16, 1), e16, jnp.int32),
                               _DNUMS, slice_sizes=(1,),
                               mode=lax.GatherScatterMode.PROMISE_IN_BOUNDS)
                e = g * 16 + e16
                for k in range(8):
                    buf[e, pl.ds(k * 16, 16)] = w * buf[e, pl.ds(k * 16, 16)]
            return 0
        lax.fori_loop(0, CH // 16, grp, 0)

    for j0 in range(3):
        fire_idx(j0, j0)
    for j0 in range(3):
        wait_idx(j0, j0)
    fire_gather(0, 0, 0, 0)
    fire_gather(0, 0, 1, 1)
    plsc.subcore_barrier()

    NJOB = 2 * NCKP

    def body(t, _):
        for v in range(12):
            q = 12 * t + v
            jc = q // 2
            vj = v // 2
            kind = v % 2
            r = v % 4
            sl = vj % 3

            rm = (r + 2) % 4
            slm = ((v - 2) % 12 // 2) % 3

            @pl.when(q >= 2)
            def _(slm=slm, rm=rm):
                wait_scatter(0, slm, rm)

            jn2 = (q + 2) // 2
            sln2 = ((vj + 1) % 3)
            kn = kind
            if kind == 1:
                jn = jc + 2
                sln = ((vj + 2) % 3)

                @pl.when((jn >= 3) & (jn < NCKP))
                def _(jn=jn, sln=sln):
                    fire_idx(jn, sln)
            else:
                @pl.when((jn2 >= 3) & (jn2 < NCKP))
                def _(jn2=jn2, sln2=sln2):
                    wait_idx(jn2, sln2)

            @pl.when(q + 2 < NJOB)
            def _(jn2=jn2, sln2=sln2, rm=rm, kn=kn):
                fire_gather(jn2, sln2, rm, kn)

            wait_gather(jc, sl, r, kind)
            if kind == 1:
                compute(sl, r)
            fire_scatter(jc, sl, r)
        return 0
    lax.fori_loop(0, NJOB // 12, body, 0)
    for qq in range(NJOB - 2, NJOB):
        vv = qq % 12
        wait_scatter(qq // 2, (vv // 2) % 3, vv % 4)
    plsc.subcore_barrier()

    rows = pl.ds(s * ROWS_PER_TILE, ROWS_PER_TILE)

    @pl.when(c == 0)
    def _():
        pltpu.sync_copy(acc_sh.at[rows], out0_hbm.at[rows])

    @pl.when(c == 1)
    def _():
        pltpu.sync_copy(acc_sh.at[rows], out1_hbm.at[rows])


_edge_pass = functools.partial(
    pl.kernel,
    out_type=[jax.ShapeDtypeStruct((NPAD, D), jnp.float32)] * 2,
    mesh=plsc.VectorSubcoreMesh(core_axis_name="c", subcore_axis_name="s"),
    scratch_types=[
        pltpu.VMEM((3, 2, CH), jnp.int32),
        pltpu.VMEM((3, CH // 16, 16), jnp.float32),
        pltpu.VMEM((CH, D), jnp.float32),
        pltpu.VMEM((CH, D), jnp.float32),
        pltpu.VMEM((CH, D), jnp.float32),
        pltpu.VMEM((CH, D), jnp.float32),
        pltpu.VMEM_SHARED((NPAD, D), jnp.float32),
        pltpu.SemaphoreType.DMA,
        pltpu.SemaphoreType.DMA,
        pltpu.SemaphoreType.DMA,
        pltpu.SemaphoreType.DMA,
        pltpu.SemaphoreType.DMA,
        pltpu.SemaphoreType.DMA,
        pltpu.SemaphoreType.DMA,
        pltpu.SemaphoreType.DMA,
        pltpu.SemaphoreType.DMA,
        pltpu.SemaphoreType.DMA,
        pltpu.SemaphoreType.DMA,
    ],
)(_edge_body)


_RB = 1000
_NB = N // _RB


def _pre_body(x_ref, wn_ref, bn_ref, clo_ref, chi_ref):
    xb = x_ref[pl.ds(pl.program_id(0) * _RB, _RB), :]
    clo_ref[...] = xb * wn_ref[...]
    chi_ref[...] = xb * bn_ref[...]


def _mid_body(a0_ref, a1_ref, h_ref, root_ref, bias_ref, wn_ref, bn_ref,
              h_out, clo_out, chi_out, *, first):
    g = a0_ref[...] + a1_ref[...] + bias_ref[...]
    if first:
        g = g + h_ref[pl.ds(pl.program_id(0) * _RB, _RB), :] * root_ref[...]
    else:
        g = g + jnp.dot(h_ref[...], root_ref[...], preferred_element_type=jnp.float32)
    g = jnp.maximum(g, 0.0)
    h_out[...] = g
    clo_out[...] = jnp.dot(g, wn_ref[...], preferred_element_type=jnp.float32)
    chi_out[...] = jnp.dot(g, bn_ref[...], preferred_element_type=jnp.float32)


def _final_body(a0_ref, a1_ref, h_ref, root_ref, bias_ref, out_ref):
    out_ref[...] = (a0_ref[...] + a1_ref[...] + bias_ref[...]
                    + jnp.dot(h_ref[...], root_ref[...], preferred_element_type=jnp.float32))


def _row_spec(width):
    return pl.BlockSpec((_RB, width), lambda i: (i, 0))


def _full_spec(r, width):
    return pl.BlockSpec((r, width), lambda i: (0, 0))


_ACC = pl.BlockSpec((_RB, D), lambda i: (i, 0))


def _pre(x, wn, bn):
    return pl.pallas_call(
        _pre_body,
        grid=(_NB,),
        in_specs=[_full_spec(N, 1), _full_spec(1, D), _full_spec(1, D)],
        out_specs=[_row_spec(D), _row_spec(D)],
        out_shape=[jax.ShapeDtypeStruct((N, D), jnp.float32)] * 2,
    )(x, wn, bn)


def _mid(acc0, acc1, h, root, bias, wn, bn, *, first):
    hw = h.shape[1]
    return pl.pallas_call(
        functools.partial(_mid_body, first=first),
        grid=(_NB,),
        in_specs=[_ACC, _ACC,
                  _full_spec(N, 1) if first else _row_spec(hw),
                  _full_spec(root.shape[0], D),
                  _full_spec(1, D), _full_spec(D, D), _full_spec(D, D)],
        out_specs=[_row_spec(D)] * 3,
        out_shape=[jax.ShapeDtypeStruct((N, D), jnp.float32)] * 3,
    )(acc0, acc1, h, root, bias, wn, bn)


def _final(acc0, acc1, h, root, bias):
    return pl.pallas_call(
        _final_body,
        grid=(_NB,),
        in_specs=[_ACC, _ACC, _row_spec(D), _full_spec(D, D), _full_spec(1, D)],
        out_specs=_row_spec(D),
        out_shape=jax.ShapeDtypeStruct((N, D), jnp.float32),
    )(acc0, acc1, h, root, bias)


def kernel(x, edge_index, edge_attribute, Wn1, bn1, root1, bias1,
           Wn2, bn2, root2, bias2, Wn3, bn3, root3, bias3):
    src = edge_index[0]
    dst = edge_index[1]
    ea = edge_attribute[:, 0]

    lanes = jnp.arange(CH, dtype=jnp.int32)
    pad_src = jnp.broadcast_to((lanes * 131) % N, (NC, NS, 1, CH))
    pad_dst = jnp.broadcast_to(10232 + (lanes % 8), (NC, NS, 1, CH))
    S = jnp.concatenate([src.reshape(NC, NS, NCHUNK, CH), pad_src], axis=2)
    T = jnp.concatenate([dst.reshape(NC, NS, NCHUNK, CH), pad_dst], axis=2)
    pki = jnp.stack([S, T], axis=3)
    pke = jnp.concatenate(
        [ea.reshape(NC, NS, NCHUNK, CH),
         jnp.zeros((NC, NS, 1, CH), jnp.float32)],
        axis=2).reshape(NC, NS, NCKP, CH // 16, 16)
    z = jnp.zeros((ROWS_PER_TILE, D), jnp.float32)

    c1lo, c1hi = _pre(x, Wn1, bn1.reshape(1, D))
    a0, a1 = _edge_pass(pki, pke, z, c1lo, c1hi)
    h1, c2lo, c2hi = _mid(a0, a1, x, root1, bias1.reshape(1, D),
                          Wn2.reshape(D, D), bn2.reshape(D, D), first=True)
    a0, a1 = _edge_pass(pki, pke, z, c2lo, c2hi)
    h2, c3lo, c3hi = _mid(a0, a1, h1, root2, bias2.reshape(1, D),
                          Wn3.reshape(D, D), bn3.reshape(D, D), first=False)
    a0, a1 = _edge_pass(pki, pke, z, c3lo, c3hi)
    return _final(a0, a1, h2, root3, bias3.reshape(1, D))
